# Initial kernel scaffold; baseline (speedup 1.0000x reference)
import jax
import jax.numpy as jnp
from jax.experimental import pallas as pl


def kernel(logits, targets, unigram):
    raise NotImplementedError("write your pallas kernel here")


# speedup vs baseline: 337.7395x; 337.7395x over previous
import functools

import jax
import jax.numpy as jnp
from jax import lax
from jax.experimental import pallas as pl
from jax.experimental.pallas import tpu as pltpu, tpu_sc as plsc

_VOCAB = 100000
_BATCH = 128
_NUM_SAMPLES = 250
_ALPHA = 0.75

_NC, _NS = 2, 16
_NW = _NC * _NS
_ROWS_PER_W = _BATCH // _NW
_SLOTS = 1024
_NVREG = _SLOTS // 16
_LIVE = 4 * _NUM_SAMPLES


def _uniform_from_hash(bits_u32):
    x = bits_u32
    x = x ^ (x >> 16)
    x = x * jnp.uint32(0x85EBCA6B)
    x = x ^ (x >> 13)
    x = x * jnp.uint32(0xC2B2AE35)
    x = x ^ (x >> 16)
    return (x >> 8).astype(jnp.int32).astype(jnp.float32) * jnp.float32(2.0**-24)


def _sc_body(cdf_hbm, p_hbm, logitsf_hbm, tpad_hbm, out_t, out_z,
             cdf_v, ids_v, lidx_v, pvals_v, lvals_v,
             tvec_v, scale_v, c_v, ut_v, tidx_v, tlog_v, zout_v, tout_v,
             sem_a, sem_b):
    wid = lax.axis_index("s") * _NC + lax.axis_index("c")
    lane = lax.iota(jnp.int32, 16)
    is_row = lane < _ROWS_PER_W

    pltpu.sync_copy(cdf_hbm, cdf_v)
    pltpu.sync_copy(tpad_hbm.at[wid], tvec_v)
    tvec = tvec_v[...]
    t = jnp.where(is_row, tvec, 0)
    rowg = jnp.where(is_row, wid * _ROWS_PER_W + lane, 0)
    tidx_v[...] = rowg * _VOCAB + t

    cp_ut = pltpu.async_copy(p_hbm.at[tvec_v], ut_v, sem_a)
    cp_tl = pltpu.async_copy(logitsf_hbm.at[tidx_v], tlog_v, sem_b)
    cp_ut.wait()
    cp_tl.wait()
    ut = ut_v[...]
    c_before = plsc.load_gather(cdf_v, [jnp.maximum(t - 1, 0)])
    c_before = jnp.where(t > 0, c_before, jnp.float32(0.0))
    scale_v[...] = jnp.float32(1.0) - ut
    c_v[...] = c_before
    ut_v[...] = ut

    m4 = lane & 3
    scale4 = plsc.load_gather(scale_v, [m4])
    c4 = plsc.load_gather(c_v, [m4])
    ut4 = plsc.load_gather(ut_v, [m4])
    rowbase4 = (wid * _ROWS_PER_W + m4) * _VOCAB
    gbase = jnp.uint32(wid * _SLOTS) + lane.astype(jnp.uint32)

    def search_group(g, _):
        for j in range(8):
            k = g * 8 + j
            bits = (gbase + (k * 16).astype(jnp.uint32)) * jnp.uint32(0x9E3779B9)
            r = _uniform_from_hash(bits)
            u = r * scale4
            u = jnp.where(u >= c4, u + ut4, u)
            c = jnp.zeros((16,), jnp.int32)
            for sh in range(16, -1, -1):
                cand = c + (1 << sh)
                gathered = plsc.load_gather(
                    cdf_v, [jnp.minimum(cand - 1, _VOCAB - 1)])
                ok = (cand <= _VOCAB) & (gathered <= u)
                c = jnp.where(ok, cand, c)
            idd = jnp.minimum(c, _VOCAB - 1)
            ids_v[g, pl.ds(j * 16, 16)] = idd
            lidx_v[g, pl.ds(j * 16, 16)] = rowbase4 + idd
        return _

    lax.fori_loop(0, 8, search_group, 0, unroll=False)

    copies = []
    for g in range(8):
        copies.append(
            pltpu.async_copy(p_hbm.at[ids_v.at[g]], pvals_v.at[g], sem_a))
        copies.append(
            pltpu.async_copy(logitsf_hbm.at[lidx_v.at[g]], lvals_v.at[g],
                             sem_b))
    for cp in copies:
        cp.wait()

    inv_n = jnp.float32(1.0 / _NUM_SAMPLES)

    def acc_group(g, acc):
        for j in range(8):
            k = g * 8 + j
            pv = pvals_v[g, pl.ds(j * 16, 16)]
            lv = lvals_v[g, pl.ds(j * 16, 16)]
            slot = k * 16 + lane
            live = slot < _LIVE
            contrib = (inv_n / pv) * jnp.exp(lv)
            acc = acc + jnp.where(live, contrib, jnp.float32(0.0))
        return acc

    acc = lax.fori_loop(0, 8, acc_group, jnp.zeros((16,), jnp.float32),
                        unroll=False)

    zvec = jnp.zeros((16,), jnp.float32)
    for r in range(_ROWS_PER_W):
        zr = jnp.sum(jnp.where(m4 == r, acc, jnp.float32(0.0)), axis=0)
        zvec = jnp.where(lane == r, zr, zvec)

    tlog = tlog_v[...]
    z_full = jnp.exp(tlog) + zvec
    zout_v[...] = jnp.where(is_row, z_full, jnp.float32(1.0))
    tout_v[...] = jnp.where(is_row, tlog, jnp.float32(0.0))
    pltpu.sync_copy(zout_v, out_z.at[wid])
    pltpu.sync_copy(tout_v, out_t.at[wid])


@jax.jit
def kernel(logits, targets, unigram):
    u = unigram.astype(jnp.float32) ** _ALPHA
    p = u / jnp.sum(u)
    cdf = jnp.cumsum(p, dtype=jnp.float32)
    logits_flat = logits.reshape(-1)
    tpad = jnp.zeros((_NW, 16), jnp.int32).at[:, :_ROWS_PER_W].set(
        targets.astype(jnp.int32).reshape(_NW, _ROWS_PER_W))

    mesh = plsc.VectorSubcoreMesh(core_axis_name="c", subcore_axis_name="s",
                                  num_cores=_NC, num_subcores=_NS)
    out_t, out_z = pl.kernel(
        _sc_body,
        out_type=[
            jax.ShapeDtypeStruct((_NW, 16), jnp.float32),
            jax.ShapeDtypeStruct((_NW, 16), jnp.float32),
        ],
        mesh=mesh,
        compiler_params=pltpu.CompilerParams(needs_layout_passes=False),
        scratch_types=[
            pltpu.VMEM((_VOCAB,), jnp.float32),
            pltpu.VMEM((8, 128), jnp.int32),
            pltpu.VMEM((8, 128), jnp.int32),
            pltpu.VMEM((8, 128), jnp.float32),
            pltpu.VMEM((8, 128), jnp.float32),
            pltpu.VMEM((16,), jnp.int32),
            pltpu.VMEM((16,), jnp.float32),
            pltpu.VMEM((16,), jnp.float32),
            pltpu.VMEM((16,), jnp.float32),
            pltpu.VMEM((16,), jnp.int32),
            pltpu.VMEM((16,), jnp.float32),
            pltpu.VMEM((16,), jnp.float32),
            pltpu.VMEM((16,), jnp.float32),
            pltpu.SemaphoreType.DMA,
            pltpu.SemaphoreType.DMA,
        ],
    )(cdf, p, logits_flat, tpad)

    tl = out_t[:, :_ROWS_PER_W].reshape(_BATCH)
    z = out_z[:, :_ROWS_PER_W].reshape(_BATCH)
    return -1.0 * jnp.mean(tl - jnp.log(z), axis=0)


# speedup vs baseline: 435.3150x; 1.2889x over previous
import jax
import jax.numpy as jnp
from jax import lax
from jax.experimental import pallas as pl
from jax.experimental.pallas import tpu as pltpu, tpu_sc as plsc

_VOCAB = 100000
_BATCH = 128
_NUM_SAMPLES = 250
_ALPHA = 0.75

_NC, _NS = 2, 16
_NW = _NC * _NS
_ROWS_PER_W = _BATCH // _NW
_SLOTS_PER_ROW = 256
_SLOTS = _ROWS_PER_W * _SLOTS_PER_ROW
_VPR = _SLOTS_PER_ROW // 16


def _uniform_from_hash(bits_u32):
    x = bits_u32
    x = x ^ (x >> 16)
    x = x * jnp.uint32(0x85EBCA6B)
    x = x ^ (x >> 13)
    x = x * jnp.uint32(0xC2B2AE35)
    x = x ^ (x >> 16)
    return (x >> 8).astype(jnp.int32).astype(jnp.float32) * jnp.float32(2.0**-24)


def _sc_body(cdf_hbm, p_hbm, logits_hbm, tpad_hbm, out_t, out_z,
             cdf_v, ids_v, pvals_v, lvals_v,
             tvec_v, scale_v, c_v, ut_v, zout_v, tout_v,
             sem_a, sem_b):
    wid = lax.axis_index("s") * _NC + lax.axis_index("c")
    lane = lax.iota(jnp.int32, 16)
    is_row = lane < _ROWS_PER_W

    pltpu.sync_copy(cdf_hbm, cdf_v)
    pltpu.sync_copy(tpad_hbm.at[wid], tvec_v)
    tvec = tvec_v[...]
    t = jnp.where(is_row, tvec, 0)

    cp_ut = pltpu.async_copy(p_hbm.at[tvec_v], ut_v, sem_a)
    cp_ut.wait()
    ut = ut_v[...]
    c_before = plsc.load_gather(cdf_v, [jnp.maximum(t - 1, 0)])
    c_before = jnp.where(t > 0, c_before, jnp.float32(0.0))
    scale_v[...] = jnp.float32(1.0) - ut
    c_v[...] = c_before
    ut_v[...] = ut

    for r in range(_ROWS_PER_W):
        rsplat = jnp.zeros((16,), jnp.int32) + r
        scale_r = plsc.load_gather(scale_v, [rsplat])
        c_r = plsc.load_gather(c_v, [rsplat])
        ut_r = plsc.load_gather(ut_v, [rsplat])
        gbase = (jnp.uint32(wid * _SLOTS + r * _SLOTS_PER_ROW)
                 + lane.astype(jnp.uint32))

        def search_pair(h, _, r=r, scale_r=scale_r, c_r=c_r, ut_r=ut_r,
                        gbase=gbase):
            for j in range(8):
                v = h * 8 + j
                bits = (gbase + (v * 16).astype(jnp.uint32)) \
                    * jnp.uint32(0x9E3779B9)
                u = _uniform_from_hash(bits) * scale_r
                u = jnp.where(u >= c_r, u + ut_r, u)
                c = jnp.zeros((16,), jnp.int32)
                for sh in range(16, -1, -1):
                    cand = c + (1 << sh)
                    gathered = plsc.load_gather(
                        cdf_v, [jnp.minimum(cand - 1, _VOCAB - 1)])
                    ok = (cand <= _VOCAB) & (gathered <= u)
                    c = jnp.where(ok, cand, c)
                idd = jnp.minimum(c, _VOCAB - 1)
                ids_v[2 * r + h, pl.ds(j * 16, 16)] = idd
            return _

        lax.fori_loop(0, 2, search_pair, 0, unroll=False)

    pcopies = [pltpu.async_copy(p_hbm.at[ids_v.at[g]], pvals_v.at[g], sem_a)
               for g in range(8)]

    tlog = jnp.zeros((16,), jnp.float32)
    for r in range(_ROWS_PER_W):
        pltpu.sync_copy(logits_hbm.at[wid * _ROWS_PER_W + r], cdf_v)
        tg = plsc.load_gather(cdf_v, [t])
        tlog = jnp.where(lane == r, tg, tlog)

        def lgather_pair(h, _, r=r):
            for j in range(8):
                idd = ids_v[2 * r + h, pl.ds(j * 16, 16)]
                lvals_v[2 * r + h, pl.ds(j * 16, 16)] = \
                    plsc.load_gather(cdf_v, [idd])
            return _

        lax.fori_loop(0, 2, lgather_pair, 0, unroll=False)

    for cp in pcopies:
        cp.wait()

    inv_n = jnp.float32(1.0 / _NUM_SAMPLES)
    zvec = jnp.zeros((16,), jnp.float32)
    for r in range(_ROWS_PER_W):
        def acc_pair(h, acc, r=r):
            for j in range(8):
                v = h * 8 + j
                pv = pvals_v[2 * r + h, pl.ds(j * 16, 16)]
                lv = lvals_v[2 * r + h, pl.ds(j * 16, 16)]
                contrib = (inv_n / pv) * jnp.exp(lv)
                live = (v * 16 + lane) < _NUM_SAMPLES
                acc = acc + jnp.where(live, contrib, jnp.float32(0.0))
            return acc

        acc = lax.fori_loop(0, 2, acc_pair, jnp.zeros((16,), jnp.float32),
                            unroll=False)
        zr = jnp.sum(acc, axis=0)
        zvec = jnp.where(lane == r, zr, zvec)

    z_full = jnp.exp(tlog) + zvec
    zout_v[...] = jnp.where(is_row, z_full, jnp.float32(1.0))
    tout_v[...] = jnp.where(is_row, tlog, jnp.float32(0.0))
    pltpu.sync_copy(zout_v, out_z.at[wid])
    pltpu.sync_copy(tout_v, out_t.at[wid])


@jax.jit
def kernel(logits, targets, unigram):
    u = unigram.astype(jnp.float32) ** _ALPHA
    p = u / jnp.sum(u)
    cdf = jnp.cumsum(p, dtype=jnp.float32)
    tpad = jnp.zeros((_NW, 16), jnp.int32).at[:, :_ROWS_PER_W].set(
        targets.astype(jnp.int32).reshape(_NW, _ROWS_PER_W))

    mesh = plsc.VectorSubcoreMesh(core_axis_name="c", subcore_axis_name="s",
                                  num_cores=_NC, num_subcores=_NS)
    out_t, out_z = pl.kernel(
        _sc_body,
        out_type=[
            jax.ShapeDtypeStruct((_NW, 16), jnp.float32),
            jax.ShapeDtypeStruct((_NW, 16), jnp.float32),
        ],
        mesh=mesh,
        compiler_params=pltpu.CompilerParams(needs_layout_passes=False),
        scratch_types=[
            pltpu.VMEM((_VOCAB,), jnp.float32),
            pltpu.VMEM((8, 128), jnp.int32),
            pltpu.VMEM((8, 128), jnp.float32),
            pltpu.VMEM((8, 128), jnp.float32),
            pltpu.VMEM((16,), jnp.int32),
            pltpu.VMEM((16,), jnp.float32),
            pltpu.VMEM((16,), jnp.float32),
            pltpu.VMEM((16,), jnp.float32),
            pltpu.VMEM((16,), jnp.float32),
            pltpu.VMEM((16,), jnp.float32),
            pltpu.SemaphoreType.DMA,
            pltpu.SemaphoreType.DMA,
        ],
    )(cdf, p, logits, tpad)

    tl = out_t[:, :_ROWS_PER_W].reshape(_BATCH)
    z = out_z[:, :_ROWS_PER_W].reshape(_BATCH)
    return -1.0 * jnp.mean(tl - jnp.log(z), axis=0)
